# baseline (speedup 1.0000x reference)
import functools

import jax
import jax.numpy as jnp
from jax import lax
from jax.experimental import pallas as pl
from jax.experimental.pallas import tpu as pltpu
from jax.experimental.pallas import tpu_sc as plsc

_S = 2048
_E = 16
_ROW_W = _S * _E
_V_LEN = (2 * _S - 1) * _E
_VW = 4080
_VR = 31
_C_SC = 14


def _sc_info():
    try:
        info = plsc.get_sparse_core_info()
        return info.num_cores, info.num_subcores
    except Exception:
        return 2, 16


@functools.cache
def _make_sc_kernel():
    nc, ns = _sc_info()
    mesh = plsc.VectorSubcoreMesh(core_axis_name="c", subcore_axis_name="s")

    @functools.partial(
        pl.kernel,
        mesh=mesh,
        out_type=jax.ShapeDtypeStruct((256, _E, _S), jnp.float32),
        scratch_types=[
            pltpu.VMEM((18, _VW), jnp.float32),
            pltpu.SemaphoreType.DMA,
        ],
        compiler_params=pltpu.CompilerParams(use_tc_tiling_on_sc=False),
    )
    def k(table_hbm, out_hbm, v3, sem):
        wid = lax.axis_index("s") * nc + lax.axis_index("c")
        min_i = 1792 + 8 * wid
        r_lo = ((_S - 1 - (min_i + 7)) * _E) >> 11

        for r in range(15):
            @pl.when(r >= r_lo)
            def _():
                pltpu.sync_copy(
                    table_hbm.at[pl.ds(2048 * r, _VW)], v3.at[r, pl.ds(0, _VW)]
                )
        pltpu.sync_copy(
            table_hbm.at[pl.ds(2048 * 15, 2048)], v3.at[15, pl.ds(0, 2048)]
        )

        last = v3[15, pl.ds(2032, _E)]

        def _fill(r, n0, count):
            def body(t, carry):
                v3[r, pl.ds(n0 + t * _E, _E)] = last
                return carry
            lax.fori_loop(0, count, body, 0)

        _fill(15, 2048, 127)
        _fill(16, 0, 255)
        @pl.when(r_lo >= 1)
        def _():
            _fill(17, 0, 255)

        copies = []
        for r in range(8):
            i = min_i + r
            off = (_S - 1 - i) * _E
            R = off >> 11
            C = pl.multiple_of(off & 2047, _E)
            copies.append(
                pltpu.make_async_copy(
                    v3.at[pl.ds(R, 16), pl.ds(C, 2048)],
                    out_hbm.at[i - 1792],
                    sem,
                )
            )
        for cp in copies:
            cp.start()
        for cp in copies:
            cp.wait()

    return k


def _build_v3b(table):
    V = jnp.concatenate(
        [table.reshape(-1), jnp.tile(table[-1], _S - 1), jnp.zeros((16,), jnp.float32)]
    )
    rows = [
        lax.dynamic_slice(V, (2048 * (q // 8 + q % 8),), (4096,))
        for q in range(8 * 24)
    ]
    return jnp.stack(rows)


def _tc_body(v3b_ref, sc_ref, out_ref):
    c = pl.program_id(0)
    pc = pl.program_id(1)

    @pl.when(c < _C_SC)
    def _():
        for g in range(16):
            i = 128 * c + 8 * pc + (g >> 1)
            K = (_S - 1 - i) * _E + (g & 1) * 16384
            Q = (K >> 11) * 8
            C = K & 2047
            slab = v3b_ref[pl.ds(pl.multiple_of(Q, 8), 8), :]
            rolled = pltpu.roll(slab, -C, axis=1)
            out_ref[0, 0, pl.ds(8 * g, 8), :] = rolled[:, :2048]

    @pl.when(c >= _C_SC)
    def _():
        for k in range(8):
            out_ref[0, 0, pl.ds(16 * k, 16), :] = sc_ref[k]


@functools.cache
def _make_tc_kernel():
    return pl.pallas_call(
        _tc_body,
        grid=(16, 16),
        in_specs=[
            pl.BlockSpec((8 * 24, 4096), lambda c, p: (0, 0)),
            pl.BlockSpec(
                (8, _E, _S),
                lambda c, p: (jnp.where(c >= _C_SC, (c - _C_SC) * 16 + p, 0), 0, 0),
            ),
        ],
        out_specs=pl.BlockSpec((1, 1, 128, 2048), lambda c, p: (0, c, p, 0)),
        out_shape=jax.ShapeDtypeStruct((1, _E, _S, _S), jnp.float32),
    )


def kernel(batch_size, seq_len, table):
    sc_part = _make_sc_kernel()(table.reshape(-1))
    return _make_tc_kernel()(_build_v3b(table), sc_part)


# speedup vs baseline: 1.0357x; 1.0357x over previous
import functools

import jax
import jax.numpy as jnp
from jax import lax
from jax.experimental import pallas as pl
from jax.experimental.pallas import tpu as pltpu
from jax.experimental.pallas import tpu_sc as plsc

_S = 2048
_E = 16
_ROW_W = _S * _E
_V_LEN = (2 * _S - 1) * _E
_VW = 4080
_VR = 31
_C_SC = 14


def _sc_info():
    try:
        info = plsc.get_sparse_core_info()
        return info.num_cores, info.num_subcores
    except Exception:
        return 2, 16


@functools.cache
def _make_sc_kernel():
    nc, ns = _sc_info()
    mesh = plsc.VectorSubcoreMesh(core_axis_name="c", subcore_axis_name="s")

    @functools.partial(
        pl.kernel,
        mesh=mesh,
        out_type=jax.ShapeDtypeStruct((256, _E, _S), jnp.float32),
        scratch_types=[
            pltpu.VMEM((18, _VW), jnp.float32),
            pltpu.SemaphoreType.DMA,
        ],
        compiler_params=pltpu.CompilerParams(use_tc_tiling_on_sc=False),
    )
    def k(table_hbm, out_hbm, v3, sem):
        wid = lax.axis_index("s") * nc + lax.axis_index("c")
        min_i = 1792 + 8 * wid
        r_lo = ((_S - 1 - (min_i + 7)) * _E) >> 11

        stage = [
            pltpu.make_async_copy(
                table_hbm.at[pl.ds(2048 * r, _VW)], v3.at[r, pl.ds(0, _VW)], sem
            )
            for r in range(15)
        ]
        stage.append(
            pltpu.make_async_copy(
                table_hbm.at[pl.ds(2048 * 15, 2048)], v3.at[15, pl.ds(0, 2048)], sem
            )
        )
        for cp in stage:
            cp.start()
        for cp in stage:
            cp.wait()

        last = v3[15, pl.ds(2032, _E)]

        def _fill(r, n0, count):
            def body(t, carry):
                v3[r, pl.ds(n0 + t * _E, _E)] = last
                return carry
            lax.fori_loop(0, count, body, 0)

        _fill(15, 2048, 127)
        _fill(16, 0, 255)
        @pl.when(r_lo >= 1)
        def _():
            _fill(17, 0, 255)

        copies = []
        for r in range(8):
            i = min_i + r
            off = (_S - 1 - i) * _E
            R = off >> 11
            C = pl.multiple_of(off & 2047, _E)
            copies.append(
                pltpu.make_async_copy(
                    v3.at[pl.ds(R, 16), pl.ds(C, 2048)],
                    out_hbm.at[i - 1792],
                    sem,
                )
            )
        for cp in copies:
            cp.start()
        for cp in copies:
            cp.wait()

    return k


def _build_v3b(table):
    V = jnp.concatenate(
        [table.reshape(-1), jnp.tile(table[-1], _S - 1), jnp.zeros((16,), jnp.float32)]
    )
    rows = [
        lax.dynamic_slice(V, (2048 * (q // 8 + q % 8),), (4096,))
        for q in range(8 * 24)
    ]
    return jnp.stack(rows)


def _tc_body(v3b_ref, sc_ref, out_ref):
    c = pl.program_id(0)
    pc = pl.program_id(1)

    @pl.when(c < _C_SC)
    def _():
        for g in range(16):
            i = 128 * c + 8 * pc + (g >> 1)
            K = (_S - 1 - i) * _E + (g & 1) * 16384
            Q = (K >> 11) * 8
            C = K & 2047
            slab = v3b_ref[pl.ds(pl.multiple_of(Q, 8), 8), :]
            rolled = pltpu.roll(slab, -C, axis=1)
            out_ref[0, 0, pl.ds(8 * g, 8), :] = rolled[:, :2048]

    @pl.when(c >= _C_SC)
    def _():
        for k in range(8):
            out_ref[0, 0, pl.ds(16 * k, 16), :] = sc_ref[k]


@functools.cache
def _make_tc_kernel():
    return pl.pallas_call(
        _tc_body,
        grid=(16, 16),
        in_specs=[
            pl.BlockSpec((8 * 24, 4096), lambda c, p: (0, 0)),
            pl.BlockSpec(
                (8, _E, _S),
                lambda c, p: (jnp.where(c >= _C_SC, (c - _C_SC) * 16 + p, 0), 0, 0),
            ),
        ],
        out_specs=pl.BlockSpec((1, 1, 128, 2048), lambda c, p: (0, c, p, 0)),
        out_shape=jax.ShapeDtypeStruct((1, _E, _S, _S), jnp.float32),
    )


def kernel(batch_size, seq_len, table):
    sc_part = _make_sc_kernel()(table.reshape(-1))
    return _make_tc_kernel()(_build_v3b(table), sc_part)


# speedup vs baseline: 1.0380x; 1.0023x over previous
import functools

import jax
import jax.numpy as jnp
from jax import lax
from jax.experimental import pallas as pl
from jax.experimental.pallas import tpu as pltpu
from jax.experimental.pallas import tpu_sc as plsc

_S = 2048
_E = 16
_ROW_W = _S * _E
_VW = 4080
_C_SC = 14


def _sc_info():
    try:
        info = plsc.get_sparse_core_info()
        return info.num_cores, info.num_subcores
    except Exception:
        return 2, 16


@functools.cache
def _make_sc_kernel():
    nc, ns = _sc_info()
    mesh = plsc.VectorSubcoreMesh(core_axis_name="c", subcore_axis_name="s")

    @functools.partial(
        pl.kernel,
        mesh=mesh,
        out_type=jax.ShapeDtypeStruct((256, _E, _S), jnp.float32),
        scratch_types=[
            pltpu.VMEM((17, _VW), jnp.float32),
            pltpu.SemaphoreType.DMA,
        ],
        compiler_params=pltpu.CompilerParams(use_tc_tiling_on_sc=False),
    )
    def k(table_hbm, out_hbm, v3, sem):
        wid = lax.axis_index("s") * nc + lax.axis_index("c")
        min_i = 1792 + 8 * wid

        stage = [
            pltpu.make_async_copy(
                table_hbm.at[pl.ds(2048 * r, _VW)], v3.at[r, pl.ds(0, _VW)], sem
            )
            for r in range(15)
        ]
        stage.append(
            pltpu.make_async_copy(
                table_hbm.at[pl.ds(2048 * 15, 2048)], v3.at[15, pl.ds(0, 2048)], sem
            )
        )
        for cp in stage:
            cp.start()
        for cp in stage:
            cp.wait()

        last = v3[15, pl.ds(2032, _E)]

        def _fill(r, n0, count):
            def body(t, carry):
                v3[r, pl.ds(n0 + t * _E, _E)] = last
                return carry
            lax.fori_loop(0, count, body, 0)

        _fill(15, 2048, 127)
        _fill(16, 0, 255)

        copies = []
        for r in range(8):
            i = min_i + r
            off = (_S - 1 - i) * _E
            R = off >> 11
            C = pl.multiple_of(off & 2047, _E)
            copies.append(
                pltpu.make_async_copy(
                    v3.at[pl.ds(R, 16), pl.ds(C, 2048)],
                    out_hbm.at[i - 1792],
                    sem,
                )
            )
        for cp in copies:
            cp.start()
        for cp in copies:
            cp.wait()

    return k


def _build_v3b(table):
    V = jnp.concatenate(
        [table.reshape(-1), jnp.tile(table[-1], _S - 1), jnp.zeros((16,), jnp.float32)]
    )
    rows = [
        lax.dynamic_slice(V, (2048 * (q // 8 + q % 8),), (4096,))
        for q in range(8 * 24)
    ]
    return jnp.stack(rows)


def _tc_body(v3b_ref, sc_ref, out_ref):
    c = pl.program_id(0)
    pc = pl.program_id(1)

    @pl.when(c < _C_SC)
    def _():
        for g in range(16):
            i = 128 * c + 8 * pc + (g >> 1)
            K = (_S - 1 - i) * _E + (g & 1) * 16384
            Q = (K >> 11) * 8
            C = K & 2047
            slab = v3b_ref[pl.ds(pl.multiple_of(Q, 8), 8), :]
            rolled = pltpu.roll(slab, -C, axis=1)
            out_ref[0, 0, pl.ds(8 * g, 8), :] = rolled[:, :2048]

    @pl.when(c >= _C_SC)
    def _():
        for k in range(8):
            out_ref[0, 0, pl.ds(16 * k, 16), :] = sc_ref[k]


@functools.cache
def _make_tc_kernel():
    return pl.pallas_call(
        _tc_body,
        grid=(16, 16),
        in_specs=[
            pl.BlockSpec((8 * 24, 4096), lambda c, p: (0, 0)),
            pl.BlockSpec(
                (8, _E, _S),
                lambda c, p: (jnp.where(c >= _C_SC, (c - _C_SC) * 16 + p, 0), 0, 0),
            ),
        ],
        out_specs=pl.BlockSpec((1, 1, 128, 2048), lambda c, p: (0, c, p, 0)),
        out_shape=jax.ShapeDtypeStruct((1, _E, _S, _S), jnp.float32),
    )


def kernel(batch_size, seq_len, table):
    sc_part = _make_sc_kernel()(table.reshape(-1))
    return _make_tc_kernel()(_build_v3b(table), sc_part)
